# baseline (speedup 1.0000x reference)
import jax
import jax.numpy as jnp
from jax.experimental import pallas as pl
from jax.experimental.pallas import tpu as pltpu

B, H, W, C = 8, 224, 224, 4
N = H * W


def _decode_body(codes_ref, counts_ref, out_ref):
    c = codes_ref[0, 0, :]
    idx = jax.lax.broadcasted_iota(jnp.int32, (N,), 0)
    valid = idx < counts_ref[0, 0, 0]
    c = jnp.where(valid, c, jnp.uint32(0))
    for j, sh in enumerate((24, 16, 8, 0)):
        ch = ((c >> jnp.uint32(sh)) & jnp.uint32(255)).astype(jnp.int32)
        out_ref[0, j, :] = ch.astype(jnp.float32) * (1.0 / 127.5) - 1.0


def kernel(images):
    x = jnp.clip((images + 1.0) * 127.5, 0.0, 255.0).astype(jnp.int32)
    flat = x.reshape(B, N, 4).astype(jnp.uint32)
    code = ((flat[..., 0] * jnp.uint32(256) + flat[..., 1]) * jnp.uint32(256)
            + flat[..., 2]) * jnp.uint32(256) + flat[..., 3]
    s = jnp.sort(code, axis=-1)
    mask = jnp.concatenate(
        [jnp.ones((B, 1), dtype=bool), s[:, 1:] != s[:, :-1]], axis=1)
    pos = jnp.cumsum(mask, axis=1) - 1
    packed = jax.vmap(lambda p, v: jnp.zeros((N,), jnp.uint32).at[p].set(v))(pos, s)
    counts = jnp.sum(mask.astype(jnp.int32), axis=1)

    out = pl.pallas_call(
        _decode_body,
        grid=(B,),
        in_specs=[
            pl.BlockSpec((1, 1, N), lambda b: (b, 0, 0)),
            pl.BlockSpec((1, 1, 1), lambda b: (b, 0, 0), memory_space=pltpu.SMEM),
        ],
        out_specs=pl.BlockSpec((1, 4, N), lambda b: (b, 0, 0)),
        out_shape=jax.ShapeDtypeStruct((B, 4, N), jnp.float32),
    )(packed.reshape(B, 1, N), counts.reshape(B, 1, 1))
    palettes = out.transpose(0, 2, 1)
    return palettes, counts


# speedup vs baseline: 4.1865x; 4.1865x over previous
import functools

import jax
import jax.numpy as jnp
from jax import lax
from jax.experimental import pallas as pl
from jax.experimental.pallas import tpu as pltpu
from jax.experimental.pallas import tpu_sc as plsc

B, H, W, C = 8, 224, 224, 4
N = H * W
NV = N // 16
NC = 2
NS = 16


def _pack_body(x_ref, out_ref):
    def q(v):
        return jnp.clip((v + 1.0) * 127.5, 0.0, 255.0).astype(jnp.int32)

    code = q(x_ref[0, 0, :])
    for j in (1, 2, 3):
        code = lax.shift_left(code, 8) | q(x_ref[0, j, :])
    out_ref[0, 0, :] = code


_sc_mesh = plsc.VectorSubcoreMesh(core_axis_name="c", subcore_axis_name="s")


@functools.partial(
    pl.kernel,
    out_type=(
        jax.ShapeDtypeStruct((B, N), jnp.int32),
        jax.ShapeDtypeStruct((B, 16), jnp.int32),
    ),
    mesh=_sc_mesh,
    compiler_params=pltpu.CompilerParams(needs_layout_passes=False),
    scratch_types=[
        pltpu.VMEM((N,), jnp.int32),
        pltpu.VMEM((N,), jnp.int32),
        pltpu.VMEM((2048,), jnp.int32),
        pltpu.VMEM((16,), jnp.int32),
    ],
)
def _sc_sort(codes_hbm, out_hbm, cnt_hbm, buf_a, buf_b, hist, cvec):
    cid = lax.axis_index("c")
    sid = lax.axis_index("s")
    b = sid * NC + cid

    @pl.when(sid < 4)
    def _():
        pltpu.sync_copy(codes_hbm.at[b], buf_a)
        iota = lax.iota(jnp.int32, 16)

        def radix_pass(src, dst, sh, radix):
            shv = jnp.full((16,), sh, jnp.int32)

            def zero(j, _):
                hist[pl.ds(j * 16, 16)] = jnp.zeros((16,), jnp.int32)
                return 0

            lax.fori_loop(0, radix // 16, zero, 0)

            def hphase(i, _):
                v = src[pl.ds(i * 16, 16)]
                d = lax.shift_right_logical(v, shv) & (radix - 1)
                rank, last = plsc.scan_count(d)
                plsc.addupdate_scatter(hist, [d], rank, mask=last)
                return 0

            lax.fori_loop(0, NV, hphase, 0)

            def sphase(j, carry):
                h = hist[pl.ds(j * 16, 16)]
                incl = plsc.cumsum(h)
                hist[pl.ds(j * 16, 16)] = incl - h + carry
                return carry + jnp.sum(h)

            lax.fori_loop(0, radix // 16, sphase, jnp.int32(0))

            def pphase(i, _):
                v = src[pl.ds(i * 16, 16)]
                d = lax.shift_right_logical(v, shv) & (radix - 1)
                rank, last = plsc.scan_count(d)
                base = plsc.load_gather(hist, [d])
                plsc.store_scatter(dst, [base + rank - 1], v)
                plsc.addupdate_scatter(hist, [d], rank, mask=last)
                return 0

            lax.fori_loop(0, NV, pphase, 0)

        radix_pass(buf_a, buf_b, 0, 2048)
        radix_pass(buf_b, buf_a, 11, 2048)
        radix_pass(buf_a, buf_b, 22, 1024)

        def dphase(i, pos):
            v = buf_b[pl.ds(i * 16, 16)]
            idxv = iota + i * 16
            pv = plsc.load_gather(buf_b, [jnp.maximum(idxv - 1, 0)])
            m = (v != pv) | (idxv == 0)
            r = plsc.cumsum(m.astype(jnp.int32))
            plsc.store_scatter(buf_a, [pos + r - 1], v, mask=m)
            return pos + jnp.sum(m.astype(jnp.int32))

        cnt = lax.fori_loop(0, NV, dphase, jnp.int32(0))
        pltpu.sync_copy(buf_a, out_hbm.at[b])
        cvec[...] = jnp.full((16,), cnt, jnp.int32)
        pltpu.sync_copy(cvec, cnt_hbm.at[b])


def _decode_body(codes_ref, counts_ref, out_ref):
    c = codes_ref[0, 0, :]
    idx = lax.broadcasted_iota(jnp.int32, (N,), 0)
    c = jnp.where(idx < counts_ref[0, 0, 0], c, jnp.int32(0))
    for j, sh in enumerate((24, 16, 8, 0)):
        ch = lax.shift_right_logical(c, jnp.int32(sh)) & 255
        out_ref[0, j, :] = ch.astype(jnp.float32) * (1.0 / 127.5) - 1.0


def kernel(images):
    xt = images.reshape(B, N, 4).transpose(0, 2, 1)
    codes = pl.pallas_call(
        _pack_body,
        grid=(B,),
        in_specs=[pl.BlockSpec((1, 4, N), lambda b: (b, 0, 0))],
        out_specs=pl.BlockSpec((1, 1, N), lambda b: (b, 0, 0)),
        out_shape=jax.ShapeDtypeStruct((B, 1, N), jnp.int32),
    )(xt)

    sorted_codes, cnt16 = _sc_sort(codes.reshape(B, N))
    counts = cnt16[:, 0]

    out = pl.pallas_call(
        _decode_body,
        grid=(B,),
        in_specs=[
            pl.BlockSpec((1, 1, N), lambda b: (b, 0, 0)),
            pl.BlockSpec((1, 1, 1), lambda b: (b, 0, 0), memory_space=pltpu.SMEM),
        ],
        out_specs=pl.BlockSpec((1, 4, N), lambda b: (b, 0, 0)),
        out_shape=jax.ShapeDtypeStruct((B, 4, N), jnp.float32),
    )(sorted_codes.reshape(B, 1, N), counts.reshape(B, 1, 1))
    palettes = out.transpose(0, 2, 1)
    return palettes, counts
